# Initial kernel scaffold; baseline (speedup 1.0000x reference)
import jax
import jax.numpy as jnp
from jax.experimental import pallas as pl


def kernel(input_docs, ner_docs_1, ner_docs_2, word_emb_mat, ner_mat_1, ner_mat_2):
    raise NotImplementedError("write your pallas kernel here")


# speedup vs baseline: 3.0902x; 3.0902x over previous
import functools

import jax
import jax.numpy as jnp
from jax import lax
from jax.experimental import pallas as pl
from jax.experimental.pallas import tpu as pltpu
from jax.experimental.pallas import tpu_sc as plsc

WORD_DIM = 128
NER_DIM_1 = 32
NER_DIM_2 = 16
OUT_DIM = WORD_DIM + NER_DIM_1 + NER_DIM_2
BATCH = 4096
SEQ = 200
N = BATCH * SEQ

NUM_CORES = 2
NUM_SUBCORES = 16
NW = NUM_CORES * NUM_SUBCORES
ROWS_PER_W = N // NW
CH = 128
N_ITERS = ROWS_PER_W // CH


@functools.partial(
    pl.kernel,
    out_type=jax.ShapeDtypeStruct((N, OUT_DIM), jnp.float32),
    mesh=plsc.VectorSubcoreMesh(
        core_axis_name="c",
        subcore_axis_name="s",
        num_cores=NUM_CORES,
        num_subcores=NUM_SUBCORES,
    ),
    compiler_params=pltpu.CompilerParams(use_tc_tiling_on_sc=False),
    scratch_types=[
        pltpu.VMEM((CH,), jnp.int32),
        pltpu.VMEM((CH,), jnp.int32),
        pltpu.VMEM((CH,), jnp.int32),
        pltpu.VMEM((CH, WORD_DIM), jnp.float32),
        pltpu.VMEM((CH, NER_DIM_1), jnp.float32),
        pltpu.VMEM((CH, NER_DIM_2), jnp.float32),
        pltpu.SemaphoreType.DMA,
    ],
)
def _emb_kernel(docs, ner1, ner2, wmat, nmat1, nmat2, out,
                idx_w, idx_1, idx_2, buf_w, buf_1, buf_2, sem):
    wid = lax.axis_index("s") * NUM_CORES + lax.axis_index("c")

    @pl.loop(0, N_ITERS)
    def _(i):
        base = wid * ROWS_PER_W + i * CH
        pltpu.sync_copy(docs.at[pl.ds(base, CH)], idx_w)
        pltpu.sync_copy(ner1.at[pl.ds(base, CH)], idx_1)
        pltpu.sync_copy(ner2.at[pl.ds(base, CH)], idx_2)
        cw = pltpu.async_copy(wmat.at[idx_w], buf_w, sem)
        c1 = pltpu.async_copy(nmat1.at[idx_1], buf_1, sem)
        c2 = pltpu.async_copy(nmat2.at[idx_2], buf_2, sem)
        cw.wait()
        c1.wait()
        c2.wait()
        pltpu.sync_copy(buf_w, out.at[pl.ds(base, CH), pl.ds(0, WORD_DIM)])
        pltpu.sync_copy(buf_1, out.at[pl.ds(base, CH), pl.ds(WORD_DIM, NER_DIM_1)])
        pltpu.sync_copy(buf_2, out.at[pl.ds(base, CH), pl.ds(WORD_DIM + NER_DIM_1, NER_DIM_2)])


def kernel(input_docs, ner_docs_1, ner_docs_2, word_emb_mat, ner_mat_1, ner_mat_2):
    docs = input_docs.reshape(N).astype(jnp.int32)
    n1 = ner_docs_1.reshape(N).astype(jnp.int32)
    n2 = ner_docs_2.reshape(N).astype(jnp.int32)
    out = _emb_kernel(docs, n1, n2, word_emb_mat, ner_mat_1, ner_mat_2)
    return out.reshape(BATCH, SEQ, OUT_DIM)


# speedup vs baseline: 3.1219x; 1.0103x over previous
import functools

import jax
import jax.numpy as jnp
from jax import lax
from jax.experimental import pallas as pl
from jax.experimental.pallas import tpu as pltpu
from jax.experimental.pallas import tpu_sc as plsc

WORD_DIM = 128
NER_DIM_1 = 32
NER_DIM_2 = 16
OUT_DIM = WORD_DIM + NER_DIM_1 + NER_DIM_2
BATCH = 4096
SEQ = 200
N = BATCH * SEQ

NUM_CORES = 2
NUM_SUBCORES = 16
NW = NUM_CORES * NUM_SUBCORES
ROWS_PER_W = N // NW
CH = 128
K = 4
CHUNKS_PER_W = ROWS_PER_W // CH
NSUPER = CHUNKS_PER_W // K


@functools.partial(
    pl.kernel,
    out_type=jax.ShapeDtypeStruct((N, OUT_DIM), jnp.float32),
    mesh=plsc.VectorSubcoreMesh(
        core_axis_name="c",
        subcore_axis_name="s",
        num_cores=NUM_CORES,
        num_subcores=NUM_SUBCORES,
    ),
    compiler_params=pltpu.CompilerParams(use_tc_tiling_on_sc=False),
    scratch_types=[
        pltpu.VMEM((K, CH), jnp.int32),
        pltpu.VMEM((K, CH), jnp.int32),
        pltpu.VMEM((K, CH), jnp.int32),
        pltpu.VMEM((K, CH, WORD_DIM), jnp.float32),
        pltpu.VMEM((K, CH, NER_DIM_1), jnp.float32),
        pltpu.VMEM((K, CH, NER_DIM_2), jnp.float32),
        pltpu.SemaphoreType.DMA((K,)),
        pltpu.SemaphoreType.DMA((K,)),
    ],
)
def _emb_kernel(docs, ner1, ner2, wmat, nmat1, nmat2, out,
                idx_w, idx_1, idx_2, buf_w, buf_1, buf_2, gsem, wsem):
    wid = lax.axis_index("s") * NUM_CORES + lax.axis_index("c")
    row0 = wid * CHUNKS_PER_W

    def write_descs(j, base):
        rows = pl.ds(base + j * CH, CH)
        return (
            pltpu.make_async_copy(
                buf_w.at[j], out.at[rows, pl.ds(0, WORD_DIM)], wsem.at[j]),
            pltpu.make_async_copy(
                buf_1.at[j], out.at[rows, pl.ds(WORD_DIM, NER_DIM_1)],
                wsem.at[j]),
            pltpu.make_async_copy(
                buf_2.at[j],
                out.at[rows, pl.ds(WORD_DIM + NER_DIM_1, NER_DIM_2)],
                wsem.at[j]),
        )

    def gather_descs(j):
        return (
            pltpu.make_async_copy(wmat.at[idx_w.at[j]], buf_w.at[j],
                                  gsem.at[j]),
            pltpu.make_async_copy(nmat1.at[idx_1.at[j]], buf_1.at[j],
                                  gsem.at[j]),
            pltpu.make_async_copy(nmat2.at[idx_2.at[j]], buf_2.at[j],
                                  gsem.at[j]),
        )

    @pl.loop(0, NSUPER)
    def _(s):
        r = row0 + s * K
        base = r * CH
        pltpu.sync_copy(docs.at[pl.ds(r, K)], idx_w)
        pltpu.sync_copy(ner1.at[pl.ds(r, K)], idx_1)
        pltpu.sync_copy(ner2.at[pl.ds(r, K)], idx_2)
        for j in range(K):
            @pl.when(s > 0)
            def _():
                for d in write_descs(j, base):
                    d.wait()

            for d in gather_descs(j):
                d.start()
        for j in range(K):
            for d in gather_descs(j):
                d.wait()
            for d in write_descs(j, base):
                d.start()

    last = (row0 + (NSUPER - 1) * K) * CH
    for j in range(K):
        for d in write_descs(j, last):
            d.wait()


def kernel(input_docs, ner_docs_1, ner_docs_2, word_emb_mat, ner_mat_1, ner_mat_2):
    docs = input_docs.reshape(N // CH, CH).astype(jnp.int32)
    n1 = ner_docs_1.reshape(N // CH, CH).astype(jnp.int32)
    n2 = ner_docs_2.reshape(N // CH, CH).astype(jnp.int32)
    out = _emb_kernel(docs, n1, n2, word_emb_mat, ner_mat_1, ner_mat_2)
    return out.reshape(BATCH, SEQ, OUT_DIM)
